# baseline (speedup 1.0000x reference)
import functools

import jax
import jax.numpy as jnp
from jax import lax
from jax.experimental import pallas as pl
from jax.experimental.pallas import tpu as pltpu
from jax.experimental.pallas import tpu_sc as plsc

N = 10000
E = 320000
D = 128
D_EDGE = 16
D_U = 32

NC = 2
NS = 16
NW = NC * NS
C = 80

EA = 163840
EB = E - EA

_mesh = plsc.VectorSubcoreMesh(core_axis_name="c", subcore_axis_name="s")


def _make_gather(e_part):
    pw = e_part // NW
    nch = pw // C
    assert nch % 4 == 1 and nch >= 9

    def body(psrc, pdst, src, dst, out, idx_s, idx_d,
             b0, b1, b2, b3, g0, g1, g2, g3, t0, t1, t2, t3):
        bufs = (b0, b1, b2, b3)
        gsem = (g0, g1, g2, g3)
        tsem = (t0, t1, t2, t3)
        wid = lax.axis_index("s") * NC + lax.axis_index("c")
        base = wid * pw
        cbase = wid * nch

        pltpu.sync_copy(src.at[pl.ds(base, pw)], idx_s)
        pltpu.sync_copy(dst.at[pl.ds(base, pw)], idx_d)

        def issue_g(k, u):
            pltpu.async_copy(
                psrc.at[idx_s.at[pl.ds(k * C, C)]], bufs[u].at[0], gsem[u])
            pltpu.async_copy(
                pdst.at[idx_d.at[pl.ds(k * C, C)]], bufs[u].at[1], gsem[u])

        def wait_g(u):
            pltpu.make_async_copy(
                psrc.at[idx_s.at[pl.ds(0, C)]], bufs[u].at[0], gsem[u]).wait()
            pltpu.make_async_copy(
                pdst.at[idx_d.at[pl.ds(0, C)]], bufs[u].at[1], gsem[u]).wait()

        def issue_st(k, u):
            pltpu.async_copy(bufs[u], out.at[cbase + k], tsem[u])

        def wait_st(u):
            pltpu.make_async_copy(bufs[u], out.at[cbase], tsem[u]).wait()

        issue_g(0, 0)
        issue_g(1, 1)
        wait_g(0)
        issue_st(0, 0)
        issue_g(2, 2)
        wait_g(1)
        issue_st(1, 1)
        issue_g(3, 3)
        wait_g(2)
        issue_st(2, 2)
        wait_st(0)
        issue_g(4, 0)
        wait_g(3)
        issue_st(3, 3)
        wait_st(1)
        issue_g(5, 1)

        def outer(i, carry):
            k0 = 4 * i
            for u in range(4):
                wait_g(u)
                issue_st(k0 + u, u)
                wait_st((u + 2) % 4)
                issue_g(k0 + u + 2, (u + 2) % 4)
            return carry

        lax.fori_loop(1, (nch - 9) // 4 + 1, outer, 0)
        for j in range(nch - 5, nch - 2):
            u = j % 4
            wait_g(u)
            issue_st(j, u)
            wait_st((u + 2) % 4)
            issue_g(j + 2, (u + 2) % 4)
        for j in range(nch - 2, nch):
            u = j % 4
            wait_g(u)
            issue_st(j, u)
        for u in range(4):
            wait_st(u)

    return pl.kernel(
        body,
        mesh=_mesh,
        out_type=jax.ShapeDtypeStruct((NW * nch, 2, C, D), jnp.float32),
        scratch_types=[
            pltpu.VMEM((pw,), jnp.int32), pltpu.VMEM((pw,), jnp.int32),
            pltpu.VMEM((2, C, D), jnp.float32),
            pltpu.VMEM((2, C, D), jnp.float32),
            pltpu.VMEM((2, C, D), jnp.float32),
            pltpu.VMEM((2, C, D), jnp.float32),
            pltpu.SemaphoreType.DMA, pltpu.SemaphoreType.DMA,
            pltpu.SemaphoreType.DMA, pltpu.SemaphoreType.DMA,
            pltpu.SemaphoreType.DMA, pltpu.SemaphoreType.DMA,
            pltpu.SemaphoreType.DMA, pltpu.SemaphoreType.DMA,
        ],
    )


_gather_full = _make_gather(E)


PW = E // NW
NCH = PW // C


def _scatter_body(eout, dsti, zeros, hpart,
                  b0, b1, b2, b3, i0, i1, i2, i3, hsh,
                  l0, l1, l2, l3, a0, a1, a2, a3):
    bufs = (b0, b1, b2, b3)
    ibufs = (i0, i1, i2, i3)
    lsem = (l0, l1, l2, l3)
    asem = (a0, a1, a2, a3)
    cid = lax.axis_index("c")
    sid = lax.axis_index("s")
    wid = sid * NC + cid
    base = wid * PW
    rz = 624
    tail = N - rz * NS

    pltpu.sync_copy(zeros.at[pl.ds(sid * rz, rz)], hsh.at[pl.ds(sid * rz, rz)])

    @pl.when(sid == 0)
    def _():
        pltpu.sync_copy(zeros.at[pl.ds(rz * NS, tail)],
                        hsh.at[pl.ds(rz * NS, tail)])

    plsc.subcore_barrier()

    def issue_ld(k, u):
        pltpu.async_copy(eout.at[pl.ds(base + k * C, C)], bufs[u], lsem[u])
        pltpu.async_copy(dsti.at[pl.ds(base + k * C, C)], ibufs[u], lsem[u])

    def wait_ld(u):
        pltpu.make_async_copy(
            eout.at[pl.ds(base, C)], bufs[u], lsem[u]).wait()
        pltpu.make_async_copy(
            dsti.at[pl.ds(base, C)], ibufs[u], lsem[u]).wait()

    def issue_sc(u):
        pltpu.async_copy(bufs[u], hsh.at[ibufs[u]], asem[u], add=True)

    def wait_sc(u):
        pltpu.make_async_copy(bufs[u], hsh.at[ibufs[u]], asem[u]).wait()

    issue_ld(0, 0)
    issue_ld(1, 1)
    wait_ld(0)
    issue_sc(0)
    issue_ld(2, 2)
    wait_ld(1)
    issue_sc(1)
    issue_ld(3, 3)
    wait_ld(2)
    issue_sc(2)
    wait_sc(0)
    issue_ld(4, 0)
    wait_ld(3)
    issue_sc(3)
    wait_sc(1)
    issue_ld(5, 1)

    def outer(i, carry):
        k0 = 4 * i
        for u in range(4):
            wait_ld(u)
            issue_sc(u)
            wait_sc((u + 2) % 4)
            issue_ld(k0 + u + 2, (u + 2) % 4)
        return carry

    lax.fori_loop(1, (NCH - 9) // 4 + 1, outer, 0)
    for j in range(NCH - 5, NCH - 2):
        u = j % 4
        wait_ld(u)
        issue_sc(u)
        wait_sc((u + 2) % 4)
        issue_ld(j + 2, (u + 2) % 4)
    for j in range(NCH - 2, NCH):
        u = j % 4
        wait_ld(u)
        issue_sc(u)
    for u in range(4):
        wait_sc(u)

    plsc.subcore_barrier()
    pltpu.sync_copy(hsh.at[pl.ds(sid * rz, rz)],
                    hpart.at[cid, pl.ds(sid * rz, rz)])

    @pl.when(sid == 0)
    def _():
        pltpu.sync_copy(hsh.at[pl.ds(rz * NS, tail)],
                        hpart.at[cid, pl.ds(rz * NS, tail)])


_scatter_call = pl.kernel(
    _scatter_body,
    mesh=_mesh,
    out_type=jax.ShapeDtypeStruct((NC, N, D), jnp.float32),
    scratch_types=[
        pltpu.VMEM((C, D), jnp.float32), pltpu.VMEM((C, D), jnp.float32),
        pltpu.VMEM((C, D), jnp.float32), pltpu.VMEM((C, D), jnp.float32),
        pltpu.VMEM((C,), jnp.int32), pltpu.VMEM((C,), jnp.int32),
        pltpu.VMEM((C,), jnp.int32), pltpu.VMEM((C,), jnp.int32),
        pltpu.VMEM_SHARED((N, D), jnp.float32),
        pltpu.SemaphoreType.DMA, pltpu.SemaphoreType.DMA,
        pltpu.SemaphoreType.DMA, pltpu.SemaphoreType.DMA,
        pltpu.SemaphoreType.DMA, pltpu.SemaphoreType.DMA,
        pltpu.SemaphoreType.DMA, pltpu.SemaphoreType.DMA,
    ],
)


_B1 = 1000
_BE = 2560
_NBA = EA // _BE


def _prep_body(nf, wsrc, wdst, g, weu, be1, wnu, bn1, psrc_o, pdst_o, ce_o, cn_o):
    nfb = nf[...]
    psrc_o[...] = jnp.dot(nfb, wsrc[...], preferred_element_type=jnp.float32)
    pdst_o[...] = jnp.dot(nfb, wdst[...], preferred_element_type=jnp.float32)

    @pl.when(pl.program_id(0) == 0)
    def _():
        gv = g[...]
        ce_o[...] = jnp.dot(gv, weu[...], preferred_element_type=jnp.float32) + be1[...]
        cn_o[...] = jnp.dot(gv, wnu[...], preferred_element_type=jnp.float32) + bn1[...]


def _edge_compute(gsgd, ef, wef, we2, ce, be2):
    gs = jnp.reshape(gsgd[:, 0, :, :], (_BE, D))
    gd = jnp.reshape(gsgd[:, 1, :, :], (_BE, D))
    pre = gs + gd + ce[...]
    pre = pre + jnp.dot(ef[...], wef[...], preferred_element_type=jnp.float32)
    r = jnp.maximum(pre, 0.0)
    return jnp.dot(r, we2[...], preferred_element_type=jnp.float32) + be2[...]


def _edge_body(gsgd, ef, wef, we2, ce, be2, out, ecol_o):
    eo = _edge_compute(gsgd, ef, wef, we2, ce, be2)
    out[...] = eo
    colsum = jnp.sum(eo, axis=0, keepdims=True)

    @pl.when(pl.program_id(0) == 0)
    def _():
        ecol_o[...] = colsum

    @pl.when(pl.program_id(0) != 0)
    def _():
        ecol_o[...] = ecol_o[...] + colsum


def _node_body(nf, hp, g, ec, wnf, wnh, cn, wn2, bn2, wun, wue, wug, bu1,
               wu2, bu2, nout_o, uout_o, comb):
    h = hp[0] + hp[1]
    pre = (jnp.dot(nf[...], wnf[...], preferred_element_type=jnp.float32)
           + jnp.dot(h, wnh[...], preferred_element_type=jnp.float32)
           + cn[...])
    r = jnp.maximum(pre, 0.0)
    nout = jnp.dot(r, wn2[...], preferred_element_type=jnp.float32) + bn2[...]
    nout_o[...] = nout
    ncol = jnp.sum(nout, axis=0, keepdims=True)

    @pl.when(pl.program_id(0) == 0)
    def _():
        comb[...] = ncol

    @pl.when(pl.program_id(0) != 0)
    def _():
        comb[...] = comb[...] + ncol

    @pl.when(pl.program_id(0) == pl.num_programs(0) - 1)
    def _():
        upre = (jnp.dot(comb[...], wun[...], preferred_element_type=jnp.float32)
                + jnp.dot(ec[...], wue[...], preferred_element_type=jnp.float32)
                + jnp.dot(g[...], wug[...], preferred_element_type=jnp.float32)
                + bu1[...])
        ur = jnp.maximum(upre, 0.0)
        uout_o[...] = jnp.dot(ur, wu2[...], preferred_element_type=jnp.float32) + bu2[...]


def _const_spec(shape):
    return pl.BlockSpec(shape, lambda i: tuple(0 for _ in shape))


_prep_call = pl.pallas_call(
    _prep_body,
    grid=(N // _B1,),
    in_specs=[
        pl.BlockSpec((_B1, D), lambda i: (i, 0)),
        _const_spec((D, D)), _const_spec((D, D)),
        _const_spec((1, D_U)), _const_spec((D_U, D)), _const_spec((1, D)),
        _const_spec((D_U, D)), _const_spec((1, D)),
    ],
    out_specs=[
        pl.BlockSpec((_B1, D), lambda i: (i, 0)),
        pl.BlockSpec((_B1, D), lambda i: (i, 0)),
        _const_spec((1, D)), _const_spec((1, D)),
    ],
    out_shape=[
        jax.ShapeDtypeStruct((N, D), jnp.float32),
        jax.ShapeDtypeStruct((N, D), jnp.float32),
        jax.ShapeDtypeStruct((1, D), jnp.float32),
        jax.ShapeDtypeStruct((1, D), jnp.float32),
    ],
)

_CPB = _BE // C

_edge_call = pl.pallas_call(
    _edge_body,
    grid=(E // _BE,),
    in_specs=[
        pl.BlockSpec((_CPB, 2, C, D), lambda i: (i, 0, 0, 0)),
        pl.BlockSpec((_BE, D_EDGE), lambda i: (i, 0)),
        _const_spec((D_EDGE, D)), _const_spec((D, D)),
        _const_spec((1, D)), _const_spec((1, D)),
    ],
    out_specs=[pl.BlockSpec((_BE, D), lambda i: (i, 0)),
               _const_spec((1, D))],
    out_shape=[jax.ShapeDtypeStruct((E, D), jnp.float32),
               jax.ShapeDtypeStruct((1, D), jnp.float32)],
)

_node_call = pl.pallas_call(
    _node_body,
    grid=(N // _B1,),
    in_specs=[
        pl.BlockSpec((_B1, D), lambda i: (i, 0)),
        pl.BlockSpec((NC, _B1, D), lambda i: (0, i, 0)),
        _const_spec((1, D_U)), _const_spec((1, D)),
        _const_spec((D, D)), _const_spec((D, D)), _const_spec((1, D)),
        _const_spec((D, D)), _const_spec((1, D)),
        _const_spec((D, D)), _const_spec((D, D)), _const_spec((D_U, D)),
        _const_spec((1, D)), _const_spec((D, D)), _const_spec((1, D)),
    ],
    out_specs=[
        pl.BlockSpec((_B1, D), lambda i: (i, 0)),
        _const_spec((1, D)),
    ],
    out_shape=[
        jax.ShapeDtypeStruct((N, D), jnp.float32),
        jax.ShapeDtypeStruct((1, D), jnp.float32),
    ],
    scratch_shapes=[pltpu.VMEM((1, D), jnp.float32)],
)


def kernel(edge_index, edge_feat, node_feat, g_repr,
           W_e1, b_e1, W_e2, b_e2, W_n1, b_n1, W_n2, b_n2,
           W_u1, b_u1, W_u2, b_u2):
    src = edge_index[0]
    dst = edge_index[1]

    W_ef = W_e1[:D_EDGE]
    W_es = W_e1[D_EDGE:D_EDGE + D]
    W_ed = W_e1[D_EDGE + D:D_EDGE + 2 * D]
    W_eu = W_e1[D_EDGE + 2 * D:]
    W_nf = W_n1[:D]
    W_nh = W_n1[D:2 * D]
    W_nu = W_n1[2 * D:]
    W_un = W_u1[:D]
    W_ue = W_u1[D:2 * D]
    W_ug = W_u1[2 * D:]

    psrc, pdst, c_e, c_n = _prep_call(
        node_feat, W_es, W_ed, g_repr, W_eu, b_e1.reshape(1, D),
        W_nu, b_n1.reshape(1, D))

    g_all = _gather_full(psrc, pdst, src, dst)

    be2 = b_e2.reshape(1, D)
    e_out, ecol = _edge_call(g_all, edge_feat, W_ef, W_e2, c_e, be2)

    hpart = _scatter_call(e_out, dst, jnp.zeros((N, D), jnp.float32))

    n_out, u_out = _node_call(
        node_feat, hpart, g_repr, ecol, W_nf, W_nh, c_n, W_n2,
        b_n2.reshape(1, D), W_un, W_ue, W_ug, b_u1.reshape(1, D), W_u2,
        b_u2.reshape(1, D))
    return (e_out, n_out, u_out)


# speedup vs baseline: 1.1231x; 1.1231x over previous
import functools

import jax
import jax.numpy as jnp
from jax import lax
from jax.experimental import pallas as pl
from jax.experimental.pallas import tpu as pltpu
from jax.experimental.pallas import tpu_sc as plsc

N = 10000
E = 320000
D = 128
D_EDGE = 16
D_U = 32

NC = 2
NS = 16
NW = NC * NS
C = 80

EA = 163840
EB = E - EA

_mesh = plsc.VectorSubcoreMesh(core_axis_name="c", subcore_axis_name="s")


def _make_gather(e_part):
    pw = e_part // NW
    nch = pw // C
    assert nch % 4 == 1 and nch >= 9

    def body(psrc, pdst, src, dst, out, idx_s, idx_d,
             b0, b1, b2, b3, g0, g1, g2, g3, t0, t1, t2, t3):
        bufs = (b0, b1, b2, b3)
        gsem = (g0, g1, g2, g3)
        tsem = (t0, t1, t2, t3)
        wid = lax.axis_index("s") * NC + lax.axis_index("c")
        base = wid * pw

        pltpu.sync_copy(src.at[pl.ds(base, pw)], idx_s)
        pltpu.sync_copy(dst.at[pl.ds(base, pw)], idx_d)

        def issue_g(k, u):
            pltpu.async_copy(
                psrc.at[idx_s.at[pl.ds(k * C, C)]], bufs[u].at[0], gsem[u])
            pltpu.async_copy(
                pdst.at[idx_d.at[pl.ds(k * C, C)]], bufs[u].at[1], gsem[u])

        def wait_g(u):
            pltpu.make_async_copy(
                psrc.at[idx_s.at[pl.ds(0, C)]], bufs[u].at[0], gsem[u]).wait()
            pltpu.make_async_copy(
                pdst.at[idx_d.at[pl.ds(0, C)]], bufs[u].at[1], gsem[u]).wait()

        def add_halves(u):
            bb = bufs[u]

            def row(r, carry):
                for g8 in range(D // 16):
                    cc = g8 * 16
                    bb[0, r, pl.ds(cc, 16)] = (bb[0, r, pl.ds(cc, 16)]
                                               + bb[1, r, pl.ds(cc, 16)])
                return carry

            lax.fori_loop(0, C, row, 0)

        def issue_st(k, u):
            pltpu.sync_copy(bufs[u].at[0], out.at[pl.ds(base + k * C, C)])

        def wait_st(u):
            del u

        issue_g(0, 0)
        issue_g(1, 1)
        wait_g(0)
        add_halves(0)
        issue_st(0, 0)
        issue_g(2, 2)
        wait_g(1)
        add_halves(1)
        issue_st(1, 1)
        issue_g(3, 3)
        wait_g(2)
        add_halves(2)
        issue_st(2, 2)
        wait_st(0)
        issue_g(4, 0)
        wait_g(3)
        add_halves(3)
        issue_st(3, 3)
        wait_st(1)
        issue_g(5, 1)

        def outer(i, carry):
            k0 = 4 * i
            for u in range(4):
                wait_g(u)
                add_halves(u)
                issue_st(k0 + u, u)
                wait_st((u + 2) % 4)
                issue_g(k0 + u + 2, (u + 2) % 4)
            return carry

        lax.fori_loop(1, (nch - 9) // 4 + 1, outer, 0)
        for j in range(nch - 5, nch - 2):
            u = j % 4
            wait_g(u)
            add_halves(u)
            issue_st(j, u)
            wait_st((u + 2) % 4)
            issue_g(j + 2, (u + 2) % 4)
        for j in range(nch - 2, nch):
            u = j % 4
            wait_g(u)
            add_halves(u)
            issue_st(j, u)
        for u in range(4):
            wait_st(u)

    return pl.kernel(
        body,
        mesh=_mesh,
        out_type=jax.ShapeDtypeStruct((e_part, D), jnp.float32),
        scratch_types=[
            pltpu.VMEM((pw,), jnp.int32), pltpu.VMEM((pw,), jnp.int32),
            pltpu.VMEM((2, C, D), jnp.float32),
            pltpu.VMEM((2, C, D), jnp.float32),
            pltpu.VMEM((2, C, D), jnp.float32),
            pltpu.VMEM((2, C, D), jnp.float32),
            pltpu.SemaphoreType.DMA, pltpu.SemaphoreType.DMA,
            pltpu.SemaphoreType.DMA, pltpu.SemaphoreType.DMA,
            pltpu.SemaphoreType.DMA, pltpu.SemaphoreType.DMA,
            pltpu.SemaphoreType.DMA, pltpu.SemaphoreType.DMA,
        ],
    )


_gather_full = _make_gather(E)


PW = E // NW
NCH = PW // C


def _scatter_body(eout, dsti, zeros, hpart,
                  b0, b1, b2, b3, i0, i1, i2, i3, hsh,
                  l0, l1, l2, l3, a0, a1, a2, a3):
    bufs = (b0, b1, b2, b3)
    ibufs = (i0, i1, i2, i3)
    lsem = (l0, l1, l2, l3)
    asem = (a0, a1, a2, a3)
    cid = lax.axis_index("c")
    sid = lax.axis_index("s")
    wid = sid * NC + cid
    base = wid * PW
    rz = 624
    tail = N - rz * NS

    pltpu.sync_copy(zeros.at[pl.ds(sid * rz, rz)], hsh.at[pl.ds(sid * rz, rz)])

    @pl.when(sid == 0)
    def _():
        pltpu.sync_copy(zeros.at[pl.ds(rz * NS, tail)],
                        hsh.at[pl.ds(rz * NS, tail)])

    plsc.subcore_barrier()

    def issue_ld(k, u):
        pltpu.async_copy(eout.at[pl.ds(base + k * C, C)], bufs[u], lsem[u])
        pltpu.async_copy(dsti.at[pl.ds(base + k * C, C)], ibufs[u], lsem[u])

    def wait_ld(u):
        pltpu.make_async_copy(
            eout.at[pl.ds(base, C)], bufs[u], lsem[u]).wait()
        pltpu.make_async_copy(
            dsti.at[pl.ds(base, C)], ibufs[u], lsem[u]).wait()

    def issue_sc(u):
        pltpu.async_copy(bufs[u], hsh.at[ibufs[u]], asem[u], add=True)

    def wait_sc(u):
        pltpu.make_async_copy(bufs[u], hsh.at[ibufs[u]], asem[u]).wait()

    issue_ld(0, 0)
    issue_ld(1, 1)
    wait_ld(0)
    issue_sc(0)
    issue_ld(2, 2)
    wait_ld(1)
    issue_sc(1)
    issue_ld(3, 3)
    wait_ld(2)
    issue_sc(2)
    wait_sc(0)
    issue_ld(4, 0)
    wait_ld(3)
    issue_sc(3)
    wait_sc(1)
    issue_ld(5, 1)

    def outer(i, carry):
        k0 = 4 * i
        for u in range(4):
            wait_ld(u)
            issue_sc(u)
            wait_sc((u + 2) % 4)
            issue_ld(k0 + u + 2, (u + 2) % 4)
        return carry

    lax.fori_loop(1, (NCH - 9) // 4 + 1, outer, 0)
    for j in range(NCH - 5, NCH - 2):
        u = j % 4
        wait_ld(u)
        issue_sc(u)
        wait_sc((u + 2) % 4)
        issue_ld(j + 2, (u + 2) % 4)
    for j in range(NCH - 2, NCH):
        u = j % 4
        wait_ld(u)
        issue_sc(u)
    for u in range(4):
        wait_sc(u)

    plsc.subcore_barrier()
    pltpu.sync_copy(hsh.at[pl.ds(sid * rz, rz)],
                    hpart.at[cid, pl.ds(sid * rz, rz)])

    @pl.when(sid == 0)
    def _():
        pltpu.sync_copy(hsh.at[pl.ds(rz * NS, tail)],
                        hpart.at[cid, pl.ds(rz * NS, tail)])


_scatter_call = pl.kernel(
    _scatter_body,
    mesh=_mesh,
    out_type=jax.ShapeDtypeStruct((NC, N, D), jnp.float32),
    scratch_types=[
        pltpu.VMEM((C, D), jnp.float32), pltpu.VMEM((C, D), jnp.float32),
        pltpu.VMEM((C, D), jnp.float32), pltpu.VMEM((C, D), jnp.float32),
        pltpu.VMEM((C,), jnp.int32), pltpu.VMEM((C,), jnp.int32),
        pltpu.VMEM((C,), jnp.int32), pltpu.VMEM((C,), jnp.int32),
        pltpu.VMEM_SHARED((N, D), jnp.float32),
        pltpu.SemaphoreType.DMA, pltpu.SemaphoreType.DMA,
        pltpu.SemaphoreType.DMA, pltpu.SemaphoreType.DMA,
        pltpu.SemaphoreType.DMA, pltpu.SemaphoreType.DMA,
        pltpu.SemaphoreType.DMA, pltpu.SemaphoreType.DMA,
    ],
)


_B1 = 1000
_BE = 2560
_NBA = EA // _BE


def _prep_body(nf, wsrc, wdst, g, weu, be1, wnu, bn1, psrc_o, pdst_o, ce_o, cn_o):
    nfb = nf[...]
    psrc_o[...] = jnp.dot(nfb, wsrc[...], preferred_element_type=jnp.float32)
    pdst_o[...] = jnp.dot(nfb, wdst[...], preferred_element_type=jnp.float32)

    @pl.when(pl.program_id(0) == 0)
    def _():
        gv = g[...]
        ce_o[...] = jnp.dot(gv, weu[...], preferred_element_type=jnp.float32) + be1[...]
        cn_o[...] = jnp.dot(gv, wnu[...], preferred_element_type=jnp.float32) + bn1[...]


def _edge_compute(gsum, ef, wef, we2, ce, be2):
    pre = gsum[...] + ce[...]
    pre = pre + jnp.dot(ef[...], wef[...], preferred_element_type=jnp.float32)
    r = jnp.maximum(pre, 0.0)
    return jnp.dot(r, we2[...], preferred_element_type=jnp.float32) + be2[...]


def _edge_body(gsum, ef, wef, we2, ce, be2, out, ecol_o):
    eo = _edge_compute(gsum, ef, wef, we2, ce, be2)
    out[...] = eo
    colsum = jnp.sum(eo, axis=0, keepdims=True)

    @pl.when(pl.program_id(0) == 0)
    def _():
        ecol_o[...] = colsum

    @pl.when(pl.program_id(0) != 0)
    def _():
        ecol_o[...] = ecol_o[...] + colsum


def _node_body(nf, hp, g, ec, wnf, wnh, cn, wn2, bn2, wun, wue, wug, bu1,
               wu2, bu2, nout_o, uout_o, comb):
    h = hp[0] + hp[1]
    pre = (jnp.dot(nf[...], wnf[...], preferred_element_type=jnp.float32)
           + jnp.dot(h, wnh[...], preferred_element_type=jnp.float32)
           + cn[...])
    r = jnp.maximum(pre, 0.0)
    nout = jnp.dot(r, wn2[...], preferred_element_type=jnp.float32) + bn2[...]
    nout_o[...] = nout
    ncol = jnp.sum(nout, axis=0, keepdims=True)

    @pl.when(pl.program_id(0) == 0)
    def _():
        comb[...] = ncol

    @pl.when(pl.program_id(0) != 0)
    def _():
        comb[...] = comb[...] + ncol

    @pl.when(pl.program_id(0) == pl.num_programs(0) - 1)
    def _():
        upre = (jnp.dot(comb[...], wun[...], preferred_element_type=jnp.float32)
                + jnp.dot(ec[...], wue[...], preferred_element_type=jnp.float32)
                + jnp.dot(g[...], wug[...], preferred_element_type=jnp.float32)
                + bu1[...])
        ur = jnp.maximum(upre, 0.0)
        uout_o[...] = jnp.dot(ur, wu2[...], preferred_element_type=jnp.float32) + bu2[...]


def _const_spec(shape):
    return pl.BlockSpec(shape, lambda i: tuple(0 for _ in shape))


_prep_call = pl.pallas_call(
    _prep_body,
    grid=(N // _B1,),
    in_specs=[
        pl.BlockSpec((_B1, D), lambda i: (i, 0)),
        _const_spec((D, D)), _const_spec((D, D)),
        _const_spec((1, D_U)), _const_spec((D_U, D)), _const_spec((1, D)),
        _const_spec((D_U, D)), _const_spec((1, D)),
    ],
    out_specs=[
        pl.BlockSpec((_B1, D), lambda i: (i, 0)),
        pl.BlockSpec((_B1, D), lambda i: (i, 0)),
        _const_spec((1, D)), _const_spec((1, D)),
    ],
    out_shape=[
        jax.ShapeDtypeStruct((N, D), jnp.float32),
        jax.ShapeDtypeStruct((N, D), jnp.float32),
        jax.ShapeDtypeStruct((1, D), jnp.float32),
        jax.ShapeDtypeStruct((1, D), jnp.float32),
    ],
)

_edge_call = pl.pallas_call(
    _edge_body,
    grid=(E // _BE,),
    in_specs=[
        pl.BlockSpec((_BE, D), lambda i: (i, 0)),
        pl.BlockSpec((_BE, D_EDGE), lambda i: (i, 0)),
        _const_spec((D_EDGE, D)), _const_spec((D, D)),
        _const_spec((1, D)), _const_spec((1, D)),
    ],
    out_specs=[pl.BlockSpec((_BE, D), lambda i: (i, 0)),
               _const_spec((1, D))],
    out_shape=[jax.ShapeDtypeStruct((E, D), jnp.float32),
               jax.ShapeDtypeStruct((1, D), jnp.float32)],
)

_node_call = pl.pallas_call(
    _node_body,
    grid=(N // _B1,),
    in_specs=[
        pl.BlockSpec((_B1, D), lambda i: (i, 0)),
        pl.BlockSpec((NC, _B1, D), lambda i: (0, i, 0)),
        _const_spec((1, D_U)), _const_spec((1, D)),
        _const_spec((D, D)), _const_spec((D, D)), _const_spec((1, D)),
        _const_spec((D, D)), _const_spec((1, D)),
        _const_spec((D, D)), _const_spec((D, D)), _const_spec((D_U, D)),
        _const_spec((1, D)), _const_spec((D, D)), _const_spec((1, D)),
    ],
    out_specs=[
        pl.BlockSpec((_B1, D), lambda i: (i, 0)),
        _const_spec((1, D)),
    ],
    out_shape=[
        jax.ShapeDtypeStruct((N, D), jnp.float32),
        jax.ShapeDtypeStruct((1, D), jnp.float32),
    ],
    scratch_shapes=[pltpu.VMEM((1, D), jnp.float32)],
)


def kernel(edge_index, edge_feat, node_feat, g_repr,
           W_e1, b_e1, W_e2, b_e2, W_n1, b_n1, W_n2, b_n2,
           W_u1, b_u1, W_u2, b_u2):
    src = edge_index[0]
    dst = edge_index[1]

    W_ef = W_e1[:D_EDGE]
    W_es = W_e1[D_EDGE:D_EDGE + D]
    W_ed = W_e1[D_EDGE + D:D_EDGE + 2 * D]
    W_eu = W_e1[D_EDGE + 2 * D:]
    W_nf = W_n1[:D]
    W_nh = W_n1[D:2 * D]
    W_nu = W_n1[2 * D:]
    W_un = W_u1[:D]
    W_ue = W_u1[D:2 * D]
    W_ug = W_u1[2 * D:]

    psrc, pdst, c_e, c_n = _prep_call(
        node_feat, W_es, W_ed, g_repr, W_eu, b_e1.reshape(1, D),
        W_nu, b_n1.reshape(1, D))

    g_all = _gather_full(psrc, pdst, src, dst)

    be2 = b_e2.reshape(1, D)
    e_out, ecol = _edge_call(g_all, edge_feat, W_ef, W_e2, c_e, be2)

    hpart = _scatter_call(e_out, dst, jnp.zeros((N, D), jnp.float32))

    n_out, u_out = _node_call(
        node_feat, hpart, g_repr, ecol, W_nf, W_nh, c_n, W_n2,
        b_n2.reshape(1, D), W_un, W_ue, W_ug, b_u1.reshape(1, D), W_u2,
        b_u2.reshape(1, D))
    return (e_out, n_out, u_out)


# speedup vs baseline: 1.1772x; 1.0481x over previous
import functools

import jax
import jax.numpy as jnp
from jax import lax
from jax.experimental import pallas as pl
from jax.experimental.pallas import tpu as pltpu
from jax.experimental.pallas import tpu_sc as plsc

N = 10000
E = 320000
D = 128
D_EDGE = 16
D_U = 32

NC = 2
NS = 16
NW = NC * NS
C = 80

EA = 163840
EB = E - EA

_mesh = plsc.VectorSubcoreMesh(core_axis_name="c", subcore_axis_name="s")


def _make_gather(e_part):
    pw = e_part // NW
    nch = pw // C
    assert nch % 4 == 1 and nch >= 9

    def body(psrc, pdst, src, dst, out, idx_s, idx_d,
             b0, b1, b2, b3, g0, g1, g2, g3, t0, t1, t2, t3):
        bufs = (b0, b1, b2, b3)
        gsem = (g0, g1, g2, g3)
        tsem = (t0, t1, t2, t3)
        wid = lax.axis_index("s") * NC + lax.axis_index("c")
        base = wid * pw

        pltpu.sync_copy(src.at[pl.ds(base, pw)], idx_s)
        pltpu.sync_copy(dst.at[pl.ds(base, pw)], idx_d)

        def issue_g(k, u):
            pltpu.async_copy(
                psrc.at[idx_s.at[pl.ds(k * C, C)]], bufs[u].at[0], gsem[u])
            pltpu.async_copy(
                pdst.at[idx_d.at[pl.ds(k * C, C)]], bufs[u].at[1], gsem[u])

        def wait_g(u):
            pltpu.make_async_copy(
                psrc.at[idx_s.at[pl.ds(0, C)]], bufs[u].at[0], gsem[u]).wait()
            pltpu.make_async_copy(
                pdst.at[idx_d.at[pl.ds(0, C)]], bufs[u].at[1], gsem[u]).wait()

        def add_halves(u):
            bb = bufs[u]

            def row(r, carry):
                for g8 in range(D // 16):
                    cc = g8 * 16
                    bb[0, r, pl.ds(cc, 16)] = (bb[0, r, pl.ds(cc, 16)]
                                               + bb[1, r, pl.ds(cc, 16)])
                return carry

            lax.fori_loop(0, C, row, 0)

        def issue_st(k, u):
            pltpu.sync_copy(bufs[u].at[0], out.at[pl.ds(base + k * C, C)])

        def wait_st(u):
            del u

        issue_g(0, 0)
        issue_g(1, 1)
        wait_g(0)
        add_halves(0)
        issue_st(0, 0)
        issue_g(2, 2)
        wait_g(1)
        add_halves(1)
        issue_st(1, 1)
        issue_g(3, 3)
        wait_g(2)
        add_halves(2)
        issue_st(2, 2)
        wait_st(0)
        issue_g(4, 0)
        wait_g(3)
        add_halves(3)
        issue_st(3, 3)
        wait_st(1)
        issue_g(5, 1)

        def outer(i, carry):
            k0 = 4 * i
            for u in range(4):
                wait_g(u)
                add_halves(u)
                issue_st(k0 + u, u)
                wait_st((u + 2) % 4)
                issue_g(k0 + u + 2, (u + 2) % 4)
            return carry

        lax.fori_loop(1, (nch - 9) // 4 + 1, outer, 0)
        for j in range(nch - 5, nch - 2):
            u = j % 4
            wait_g(u)
            add_halves(u)
            issue_st(j, u)
            wait_st((u + 2) % 4)
            issue_g(j + 2, (u + 2) % 4)
        for j in range(nch - 2, nch):
            u = j % 4
            wait_g(u)
            add_halves(u)
            issue_st(j, u)
        for u in range(4):
            wait_st(u)

    return pl.kernel(
        body,
        mesh=_mesh,
        out_type=jax.ShapeDtypeStruct((e_part, D), jnp.float32),
        scratch_types=[
            pltpu.VMEM((pw,), jnp.int32), pltpu.VMEM((pw,), jnp.int32),
            pltpu.VMEM((2, C, D), jnp.float32),
            pltpu.VMEM((2, C, D), jnp.float32),
            pltpu.VMEM((2, C, D), jnp.float32),
            pltpu.VMEM((2, C, D), jnp.float32),
            pltpu.SemaphoreType.DMA, pltpu.SemaphoreType.DMA,
            pltpu.SemaphoreType.DMA, pltpu.SemaphoreType.DMA,
            pltpu.SemaphoreType.DMA, pltpu.SemaphoreType.DMA,
            pltpu.SemaphoreType.DMA, pltpu.SemaphoreType.DMA,
        ],
    )


_gather_full = _make_gather(E)


PW = E // NW
NCH = PW // C


def _scatter_body(eout, dsti, zeros, hpart,
                  b0, b1, b2, b3, i0, i1, i2, i3, hsh,
                  l0, l1, l2, l3, a0, a1, a2, a3):
    bufs = (b0, b1, b2, b3)
    ibufs = (i0, i1, i2, i3)
    lsem = (l0, l1, l2, l3)
    asem = (a0, a1, a2, a3)
    cid = lax.axis_index("c")
    sid = lax.axis_index("s")
    wid = sid * NC + cid
    base = wid * PW
    rz = 624
    tail = N - rz * NS

    pltpu.sync_copy(zeros.at[pl.ds(sid * rz, rz)], hsh.at[pl.ds(sid * rz, rz)])

    @pl.when(sid == 0)
    def _():
        pltpu.sync_copy(zeros.at[pl.ds(rz * NS, tail)],
                        hsh.at[pl.ds(rz * NS, tail)])

    plsc.subcore_barrier()

    def issue_ld(k, u):
        pltpu.async_copy(eout.at[pl.ds(base + k * C, C)], bufs[u], lsem[u])
        pltpu.async_copy(dsti.at[pl.ds(base + k * C, C)], ibufs[u], lsem[u])

    def wait_ld(u):
        pltpu.make_async_copy(
            eout.at[pl.ds(base, C)], bufs[u], lsem[u]).wait()
        pltpu.make_async_copy(
            dsti.at[pl.ds(base, C)], ibufs[u], lsem[u]).wait()

    def issue_sc(u):
        pltpu.async_copy(bufs[u], hsh.at[ibufs[u]], asem[u], add=True)

    def wait_sc(u):
        pltpu.make_async_copy(bufs[u], hsh.at[ibufs[u]], asem[u]).wait()

    issue_ld(0, 0)
    issue_ld(1, 1)
    wait_ld(0)
    issue_sc(0)
    issue_ld(2, 2)
    wait_ld(1)
    issue_sc(1)
    issue_ld(3, 3)
    wait_ld(2)
    issue_sc(2)
    wait_sc(0)
    issue_ld(4, 0)
    wait_ld(3)
    issue_sc(3)
    wait_sc(1)
    issue_ld(5, 1)

    def outer(i, carry):
        k0 = 4 * i
        for u in range(4):
            wait_ld(u)
            issue_sc(u)
            wait_sc((u + 2) % 4)
            issue_ld(k0 + u + 2, (u + 2) % 4)
        return carry

    lax.fori_loop(1, (NCH - 9) // 4 + 1, outer, 0)
    for j in range(NCH - 5, NCH - 2):
        u = j % 4
        wait_ld(u)
        issue_sc(u)
        wait_sc((u + 2) % 4)
        issue_ld(j + 2, (u + 2) % 4)
    for j in range(NCH - 2, NCH):
        u = j % 4
        wait_ld(u)
        issue_sc(u)
    for u in range(4):
        wait_sc(u)

    plsc.subcore_barrier()
    pltpu.sync_copy(hsh.at[pl.ds(sid * rz, rz)],
                    hpart.at[cid, pl.ds(sid * rz, rz)])

    @pl.when(sid == 0)
    def _():
        pltpu.sync_copy(hsh.at[pl.ds(rz * NS, tail)],
                        hpart.at[cid, pl.ds(rz * NS, tail)])


_scatter_call = pl.kernel(
    _scatter_body,
    mesh=_mesh,
    out_type=jax.ShapeDtypeStruct((NC, N, D), jnp.float32),
    scratch_types=[
        pltpu.VMEM((C, D), jnp.float32), pltpu.VMEM((C, D), jnp.float32),
        pltpu.VMEM((C, D), jnp.float32), pltpu.VMEM((C, D), jnp.float32),
        pltpu.VMEM((C,), jnp.int32), pltpu.VMEM((C,), jnp.int32),
        pltpu.VMEM((C,), jnp.int32), pltpu.VMEM((C,), jnp.int32),
        pltpu.VMEM_SHARED((N, D), jnp.float32),
        pltpu.SemaphoreType.DMA, pltpu.SemaphoreType.DMA,
        pltpu.SemaphoreType.DMA, pltpu.SemaphoreType.DMA,
        pltpu.SemaphoreType.DMA, pltpu.SemaphoreType.DMA,
        pltpu.SemaphoreType.DMA, pltpu.SemaphoreType.DMA,
    ],
)


_B1 = 1000
_BE = 4000
_NBA = EA // _BE


def _prep_body(nf, wsrc, wdst, g, weu, be1, wnu, bn1, psrc_o, pdst_o, ce_o, cn_o):
    nfb = nf[...]
    psrc_o[...] = jnp.dot(nfb, wsrc[...], preferred_element_type=jnp.float32)
    pdst_o[...] = jnp.dot(nfb, wdst[...], preferred_element_type=jnp.float32)

    @pl.when(pl.program_id(0) == 0)
    def _():
        gv = g[...]
        ce_o[...] = jnp.dot(gv, weu[...], preferred_element_type=jnp.float32) + be1[...]
        cn_o[...] = jnp.dot(gv, wnu[...], preferred_element_type=jnp.float32) + bn1[...]


def _edge_compute(gsum, ef, wef, we2, ce, be2):
    pre = gsum[...] + ce[...]
    pre = pre + jnp.dot(ef[...], wef[...], preferred_element_type=jnp.float32)
    r = jnp.maximum(pre, 0.0)
    return jnp.dot(r, we2[...], preferred_element_type=jnp.float32) + be2[...]


def _edge_body(gsum, ef, wef, we2, ce, be2, out, ecol_o):
    eo = _edge_compute(gsum, ef, wef, we2, ce, be2)
    out[...] = eo
    colsum = jnp.sum(eo, axis=0, keepdims=True)

    @pl.when(pl.program_id(0) == 0)
    def _():
        ecol_o[...] = colsum

    @pl.when(pl.program_id(0) != 0)
    def _():
        ecol_o[...] = ecol_o[...] + colsum


def _node_body(nf, hp, g, ec, wnf, wnh, cn, wn2, bn2, wun, wue, wug, bu1,
               wu2, bu2, nout_o, uout_o, comb):
    h = hp[0] + hp[1]
    pre = (jnp.dot(nf[...], wnf[...], preferred_element_type=jnp.float32)
           + jnp.dot(h, wnh[...], preferred_element_type=jnp.float32)
           + cn[...])
    r = jnp.maximum(pre, 0.0)
    nout = jnp.dot(r, wn2[...], preferred_element_type=jnp.float32) + bn2[...]
    nout_o[...] = nout
    ncol = jnp.sum(nout, axis=0, keepdims=True)

    @pl.when(pl.program_id(0) == 0)
    def _():
        comb[...] = ncol

    @pl.when(pl.program_id(0) != 0)
    def _():
        comb[...] = comb[...] + ncol

    @pl.when(pl.program_id(0) == pl.num_programs(0) - 1)
    def _():
        upre = (jnp.dot(comb[...], wun[...], preferred_element_type=jnp.float32)
                + jnp.dot(ec[...], wue[...], preferred_element_type=jnp.float32)
                + jnp.dot(g[...], wug[...], preferred_element_type=jnp.float32)
                + bu1[...])
        ur = jnp.maximum(upre, 0.0)
        uout_o[...] = jnp.dot(ur, wu2[...], preferred_element_type=jnp.float32) + bu2[...]


def _const_spec(shape):
    return pl.BlockSpec(shape, lambda i: tuple(0 for _ in shape))


_prep_call = pl.pallas_call(
    _prep_body,
    grid=(N // _B1,),
    in_specs=[
        pl.BlockSpec((_B1, D), lambda i: (i, 0)),
        _const_spec((D, D)), _const_spec((D, D)),
        _const_spec((1, D_U)), _const_spec((D_U, D)), _const_spec((1, D)),
        _const_spec((D_U, D)), _const_spec((1, D)),
    ],
    out_specs=[
        pl.BlockSpec((_B1, D), lambda i: (i, 0)),
        pl.BlockSpec((_B1, D), lambda i: (i, 0)),
        _const_spec((1, D)), _const_spec((1, D)),
    ],
    out_shape=[
        jax.ShapeDtypeStruct((N, D), jnp.float32),
        jax.ShapeDtypeStruct((N, D), jnp.float32),
        jax.ShapeDtypeStruct((1, D), jnp.float32),
        jax.ShapeDtypeStruct((1, D), jnp.float32),
    ],
)

_edge_call = pl.pallas_call(
    _edge_body,
    grid=(E // _BE,),
    in_specs=[
        pl.BlockSpec((_BE, D), lambda i: (i, 0)),
        pl.BlockSpec((_BE, D_EDGE), lambda i: (i, 0)),
        _const_spec((D_EDGE, D)), _const_spec((D, D)),
        _const_spec((1, D)), _const_spec((1, D)),
    ],
    out_specs=[pl.BlockSpec((_BE, D), lambda i: (i, 0)),
               _const_spec((1, D))],
    out_shape=[jax.ShapeDtypeStruct((E, D), jnp.float32),
               jax.ShapeDtypeStruct((1, D), jnp.float32)],
)

_node_call = pl.pallas_call(
    _node_body,
    grid=(N // _B1,),
    in_specs=[
        pl.BlockSpec((_B1, D), lambda i: (i, 0)),
        pl.BlockSpec((NC, _B1, D), lambda i: (0, i, 0)),
        _const_spec((1, D_U)), _const_spec((1, D)),
        _const_spec((D, D)), _const_spec((D, D)), _const_spec((1, D)),
        _const_spec((D, D)), _const_spec((1, D)),
        _const_spec((D, D)), _const_spec((D, D)), _const_spec((D_U, D)),
        _const_spec((1, D)), _const_spec((D, D)), _const_spec((1, D)),
    ],
    out_specs=[
        pl.BlockSpec((_B1, D), lambda i: (i, 0)),
        _const_spec((1, D)),
    ],
    out_shape=[
        jax.ShapeDtypeStruct((N, D), jnp.float32),
        jax.ShapeDtypeStruct((1, D), jnp.float32),
    ],
    scratch_shapes=[pltpu.VMEM((1, D), jnp.float32)],
)


def kernel(edge_index, edge_feat, node_feat, g_repr,
           W_e1, b_e1, W_e2, b_e2, W_n1, b_n1, W_n2, b_n2,
           W_u1, b_u1, W_u2, b_u2):
    src = edge_index[0]
    dst = edge_index[1]

    W_ef = W_e1[:D_EDGE]
    W_es = W_e1[D_EDGE:D_EDGE + D]
    W_ed = W_e1[D_EDGE + D:D_EDGE + 2 * D]
    W_eu = W_e1[D_EDGE + 2 * D:]
    W_nf = W_n1[:D]
    W_nh = W_n1[D:2 * D]
    W_nu = W_n1[2 * D:]
    W_un = W_u1[:D]
    W_ue = W_u1[D:2 * D]
    W_ug = W_u1[2 * D:]

    psrc, pdst, c_e, c_n = _prep_call(
        node_feat, W_es, W_ed, g_repr, W_eu, b_e1.reshape(1, D),
        W_nu, b_n1.reshape(1, D))

    g_all = _gather_full(psrc, pdst, src, dst)

    be2 = b_e2.reshape(1, D)
    e_out, ecol = _edge_call(g_all, edge_feat, W_ef, W_e2, c_e, be2)

    hpart = _scatter_call(e_out, dst, jnp.zeros((N, D), jnp.float32))

    n_out, u_out = _node_call(
        node_feat, hpart, g_repr, ecol, W_nf, W_nh, c_n, W_n2,
        b_n2.reshape(1, D), W_un, W_ue, W_ug, b_u1.reshape(1, D), W_u2,
        b_u2.reshape(1, D))
    return (e_out, n_out, u_out)


# speedup vs baseline: 1.2170x; 1.0339x over previous
import functools

import jax
import jax.numpy as jnp
from jax import lax
from jax.experimental import pallas as pl
from jax.experimental.pallas import tpu as pltpu
from jax.experimental.pallas import tpu_sc as plsc

N = 10000
E = 320000
D = 128
D_EDGE = 16
D_U = 32

NC = 2
NS = 16
NW = NC * NS
C = 80

EA = 163840
EB = E - EA

_mesh = plsc.VectorSubcoreMesh(core_axis_name="c", subcore_axis_name="s")


def _make_gather(e_part):
    pw = e_part // NW
    nch = pw // C
    assert nch % 4 == 1 and nch >= 9

    def body(psrc, pdst, src, dst, out, idx_s, idx_d,
             b0, b1, b2, b3, g0, g1, g2, g3, t0, t1, t2, t3):
        bufs = (b0, b1, b2, b3)
        gsem = (g0, g1, g2, g3)
        tsem = (t0, t1, t2, t3)
        wid = lax.axis_index("s") * NC + lax.axis_index("c")
        base = wid * pw

        pltpu.sync_copy(src.at[pl.ds(base, pw)], idx_s)
        pltpu.sync_copy(dst.at[pl.ds(base, pw)], idx_d)

        def issue_g(k, u):
            pltpu.async_copy(
                psrc.at[idx_s.at[pl.ds(k * C, C)]], bufs[u].at[0], gsem[u])
            pltpu.async_copy(
                pdst.at[idx_d.at[pl.ds(k * C, C)]], bufs[u].at[1], gsem[u])

        def wait_g(u):
            pltpu.make_async_copy(
                psrc.at[idx_s.at[pl.ds(0, C)]], bufs[u].at[0], gsem[u]).wait()
            pltpu.make_async_copy(
                pdst.at[idx_d.at[pl.ds(0, C)]], bufs[u].at[1], gsem[u]).wait()

        def add_halves(u):
            bb = bufs[u]

            def row(r, carry):
                for g8 in range(D // 16):
                    cc = g8 * 16
                    bb[0, r, pl.ds(cc, 16)] = (bb[0, r, pl.ds(cc, 16)]
                                               + bb[1, r, pl.ds(cc, 16)])
                return carry

            lax.fori_loop(0, C, row, 0)

        def issue_st(k, u):
            pltpu.sync_copy(bufs[u].at[0], out.at[pl.ds(base + k * C, C)])

        def wait_st(u):
            del u

        issue_g(0, 0)
        issue_g(1, 1)
        wait_g(0)
        add_halves(0)
        issue_st(0, 0)
        issue_g(2, 2)
        wait_g(1)
        add_halves(1)
        issue_st(1, 1)
        issue_g(3, 3)
        wait_g(2)
        add_halves(2)
        issue_st(2, 2)
        wait_st(0)
        issue_g(4, 0)
        wait_g(3)
        add_halves(3)
        issue_st(3, 3)
        wait_st(1)
        issue_g(5, 1)

        def outer(i, carry):
            k0 = 4 * i
            for u in range(4):
                wait_g(u)
                add_halves(u)
                issue_st(k0 + u, u)
                wait_st((u + 2) % 4)
                issue_g(k0 + u + 2, (u + 2) % 4)
            return carry

        lax.fori_loop(1, (nch - 9) // 4 + 1, outer, 0)
        for j in range(nch - 5, nch - 2):
            u = j % 4
            wait_g(u)
            add_halves(u)
            issue_st(j, u)
            wait_st((u + 2) % 4)
            issue_g(j + 2, (u + 2) % 4)
        for j in range(nch - 2, nch):
            u = j % 4
            wait_g(u)
            add_halves(u)
            issue_st(j, u)
        for u in range(4):
            wait_st(u)

    return pl.kernel(
        body,
        mesh=_mesh,
        out_type=jax.ShapeDtypeStruct((e_part, D), jnp.float32),
        scratch_types=[
            pltpu.VMEM((pw,), jnp.int32), pltpu.VMEM((pw,), jnp.int32),
            pltpu.VMEM((2, C, D), jnp.float32),
            pltpu.VMEM((2, C, D), jnp.float32),
            pltpu.VMEM((2, C, D), jnp.float32),
            pltpu.VMEM((2, C, D), jnp.float32),
            pltpu.SemaphoreType.DMA, pltpu.SemaphoreType.DMA,
            pltpu.SemaphoreType.DMA, pltpu.SemaphoreType.DMA,
            pltpu.SemaphoreType.DMA, pltpu.SemaphoreType.DMA,
            pltpu.SemaphoreType.DMA, pltpu.SemaphoreType.DMA,
        ],
    )


_gather_full = _make_gather(E)


PW = E // NW
NCH = PW // C


def _scatter_body(eout, dsti, zeros, hpart,
                  b0, b1, b2, b3, i0, i1, i2, i3, hsh,
                  l0, l1, l2, l3, a0, a1, a2, a3):
    bufs = (b0, b1, b2, b3)
    ibufs = (i0, i1, i2, i3)
    lsem = (l0, l1, l2, l3)
    asem = (a0, a1, a2, a3)
    cid = lax.axis_index("c")
    sid = lax.axis_index("s")
    wid = sid * NC + cid
    base = wid * PW
    rz = 624
    tail = N - rz * NS

    pltpu.sync_copy(zeros.at[pl.ds(sid * rz, rz)], hsh.at[pl.ds(sid * rz, rz)])

    @pl.when(sid == 0)
    def _():
        pltpu.sync_copy(zeros.at[pl.ds(rz * NS, tail)],
                        hsh.at[pl.ds(rz * NS, tail)])

    plsc.subcore_barrier()

    def issue_ld(k, u):
        pltpu.async_copy(eout.at[pl.ds(base + k * C, C)], bufs[u], lsem[u])
        pltpu.async_copy(dsti.at[pl.ds(base + k * C, C)], ibufs[u], lsem[u])

    def wait_ld(u):
        pltpu.make_async_copy(
            eout.at[pl.ds(base, C)], bufs[u], lsem[u]).wait()
        pltpu.make_async_copy(
            dsti.at[pl.ds(base, C)], ibufs[u], lsem[u]).wait()

    def issue_sc(u):
        pltpu.async_copy(bufs[u], hsh.at[ibufs[u]], asem[u], add=True)

    def wait_sc(u):
        pltpu.make_async_copy(bufs[u], hsh.at[ibufs[u]], asem[u]).wait()

    issue_ld(0, 0)
    issue_ld(1, 1)
    wait_ld(0)
    issue_sc(0)
    issue_ld(2, 2)
    wait_ld(1)
    issue_sc(1)
    issue_ld(3, 3)
    wait_ld(2)
    issue_sc(2)
    wait_sc(0)
    issue_ld(4, 0)
    wait_ld(3)
    issue_sc(3)
    wait_sc(1)
    issue_ld(5, 1)

    def outer(i, carry):
        k0 = 4 * i
        for u in range(4):
            wait_ld(u)
            issue_sc(u)
            wait_sc((u + 2) % 4)
            issue_ld(k0 + u + 2, (u + 2) % 4)
        return carry

    lax.fori_loop(1, (NCH - 9) // 4 + 1, outer, 0)
    for j in range(NCH - 5, NCH - 2):
        u = j % 4
        wait_ld(u)
        issue_sc(u)
        wait_sc((u + 2) % 4)
        issue_ld(j + 2, (u + 2) % 4)
    for j in range(NCH - 2, NCH):
        u = j % 4
        wait_ld(u)
        issue_sc(u)
    for u in range(4):
        wait_sc(u)

    plsc.subcore_barrier()
    pltpu.sync_copy(hsh.at[pl.ds(sid * rz, rz)],
                    hpart.at[cid, pl.ds(sid * rz, rz)])

    @pl.when(sid == 0)
    def _():
        pltpu.sync_copy(hsh.at[pl.ds(rz * NS, tail)],
                        hpart.at[cid, pl.ds(rz * NS, tail)])


_scatter_call = pl.kernel(
    _scatter_body,
    mesh=_mesh,
    out_type=jax.ShapeDtypeStruct((NC, N, D), jnp.float32),
    scratch_types=[
        pltpu.VMEM((C, D), jnp.float32), pltpu.VMEM((C, D), jnp.float32),
        pltpu.VMEM((C, D), jnp.float32), pltpu.VMEM((C, D), jnp.float32),
        pltpu.VMEM((C,), jnp.int32), pltpu.VMEM((C,), jnp.int32),
        pltpu.VMEM((C,), jnp.int32), pltpu.VMEM((C,), jnp.int32),
        pltpu.VMEM_SHARED((N, D), jnp.float32),
        pltpu.SemaphoreType.DMA, pltpu.SemaphoreType.DMA,
        pltpu.SemaphoreType.DMA, pltpu.SemaphoreType.DMA,
        pltpu.SemaphoreType.DMA, pltpu.SemaphoreType.DMA,
        pltpu.SemaphoreType.DMA, pltpu.SemaphoreType.DMA,
    ],
)


_B1 = 2000
_BE = 8000
_NBA = EA // _BE


def _prep_body(nf, wsrc, wdst, g, weu, be1, wnu, bn1, psrc_o, pdst_o, ce_o, cn_o):
    nfb = nf[...]
    psrc_o[...] = jnp.dot(nfb, wsrc[...], preferred_element_type=jnp.float32)
    pdst_o[...] = jnp.dot(nfb, wdst[...], preferred_element_type=jnp.float32)

    @pl.when(pl.program_id(0) == 0)
    def _():
        gv = g[...]
        ce_o[...] = jnp.dot(gv, weu[...], preferred_element_type=jnp.float32) + be1[...]
        cn_o[...] = jnp.dot(gv, wnu[...], preferred_element_type=jnp.float32) + bn1[...]


def _edge_compute(gsum, ef, wef, we2, ce, be2):
    pre = gsum[...] + ce[...]
    pre = pre + jnp.dot(ef[...], wef[...], preferred_element_type=jnp.float32)
    r = jnp.maximum(pre, 0.0)
    return jnp.dot(r, we2[...], preferred_element_type=jnp.float32) + be2[...]


def _edge_body(gsum, ef, wef, we2, ce, be2, out, ecol_o):
    eo = _edge_compute(gsum, ef, wef, we2, ce, be2)
    out[...] = eo
    colsum = jnp.sum(eo, axis=0, keepdims=True)

    @pl.when(pl.program_id(0) == 0)
    def _():
        ecol_o[...] = colsum

    @pl.when(pl.program_id(0) != 0)
    def _():
        ecol_o[...] = ecol_o[...] + colsum


def _node_body(nf, hp, g, ec, wnf, wnh, cn, wn2, bn2, wun, wue, wug, bu1,
               wu2, bu2, nout_o, uout_o, comb):
    h = hp[0] + hp[1]
    pre = (jnp.dot(nf[...], wnf[...], preferred_element_type=jnp.float32)
           + jnp.dot(h, wnh[...], preferred_element_type=jnp.float32)
           + cn[...])
    r = jnp.maximum(pre, 0.0)
    nout = jnp.dot(r, wn2[...], preferred_element_type=jnp.float32) + bn2[...]
    nout_o[...] = nout
    ncol = jnp.sum(nout, axis=0, keepdims=True)

    @pl.when(pl.program_id(0) == 0)
    def _():
        comb[...] = ncol

    @pl.when(pl.program_id(0) != 0)
    def _():
        comb[...] = comb[...] + ncol

    @pl.when(pl.program_id(0) == pl.num_programs(0) - 1)
    def _():
        upre = (jnp.dot(comb[...], wun[...], preferred_element_type=jnp.float32)
                + jnp.dot(ec[...], wue[...], preferred_element_type=jnp.float32)
                + jnp.dot(g[...], wug[...], preferred_element_type=jnp.float32)
                + bu1[...])
        ur = jnp.maximum(upre, 0.0)
        uout_o[...] = jnp.dot(ur, wu2[...], preferred_element_type=jnp.float32) + bu2[...]


def _const_spec(shape):
    return pl.BlockSpec(shape, lambda i: tuple(0 for _ in shape))


_prep_call = pl.pallas_call(
    _prep_body,
    grid=(N // _B1,),
    in_specs=[
        pl.BlockSpec((_B1, D), lambda i: (i, 0)),
        _const_spec((D, D)), _const_spec((D, D)),
        _const_spec((1, D_U)), _const_spec((D_U, D)), _const_spec((1, D)),
        _const_spec((D_U, D)), _const_spec((1, D)),
    ],
    out_specs=[
        pl.BlockSpec((_B1, D), lambda i: (i, 0)),
        pl.BlockSpec((_B1, D), lambda i: (i, 0)),
        _const_spec((1, D)), _const_spec((1, D)),
    ],
    out_shape=[
        jax.ShapeDtypeStruct((N, D), jnp.float32),
        jax.ShapeDtypeStruct((N, D), jnp.float32),
        jax.ShapeDtypeStruct((1, D), jnp.float32),
        jax.ShapeDtypeStruct((1, D), jnp.float32),
    ],
)

_edge_call = pl.pallas_call(
    _edge_body,
    grid=(E // _BE,),
    in_specs=[
        pl.BlockSpec((_BE, D), lambda i: (i, 0)),
        pl.BlockSpec((_BE, D_EDGE), lambda i: (i, 0)),
        _const_spec((D_EDGE, D)), _const_spec((D, D)),
        _const_spec((1, D)), _const_spec((1, D)),
    ],
    out_specs=[pl.BlockSpec((_BE, D), lambda i: (i, 0)),
               _const_spec((1, D))],
    out_shape=[jax.ShapeDtypeStruct((E, D), jnp.float32),
               jax.ShapeDtypeStruct((1, D), jnp.float32)],
)

_node_call = pl.pallas_call(
    _node_body,
    grid=(N // _B1,),
    in_specs=[
        pl.BlockSpec((_B1, D), lambda i: (i, 0)),
        pl.BlockSpec((NC, _B1, D), lambda i: (0, i, 0)),
        _const_spec((1, D_U)), _const_spec((1, D)),
        _const_spec((D, D)), _const_spec((D, D)), _const_spec((1, D)),
        _const_spec((D, D)), _const_spec((1, D)),
        _const_spec((D, D)), _const_spec((D, D)), _const_spec((D_U, D)),
        _const_spec((1, D)), _const_spec((D, D)), _const_spec((1, D)),
    ],
    out_specs=[
        pl.BlockSpec((_B1, D), lambda i: (i, 0)),
        _const_spec((1, D)),
    ],
    out_shape=[
        jax.ShapeDtypeStruct((N, D), jnp.float32),
        jax.ShapeDtypeStruct((1, D), jnp.float32),
    ],
    scratch_shapes=[pltpu.VMEM((1, D), jnp.float32)],
)


def kernel(edge_index, edge_feat, node_feat, g_repr,
           W_e1, b_e1, W_e2, b_e2, W_n1, b_n1, W_n2, b_n2,
           W_u1, b_u1, W_u2, b_u2):
    src = edge_index[0]
    dst = edge_index[1]

    W_ef = W_e1[:D_EDGE]
    W_es = W_e1[D_EDGE:D_EDGE + D]
    W_ed = W_e1[D_EDGE + D:D_EDGE + 2 * D]
    W_eu = W_e1[D_EDGE + 2 * D:]
    W_nf = W_n1[:D]
    W_nh = W_n1[D:2 * D]
    W_nu = W_n1[2 * D:]
    W_un = W_u1[:D]
    W_ue = W_u1[D:2 * D]
    W_ug = W_u1[2 * D:]

    psrc, pdst, c_e, c_n = _prep_call(
        node_feat, W_es, W_ed, g_repr, W_eu, b_e1.reshape(1, D),
        W_nu, b_n1.reshape(1, D))

    g_all = _gather_full(psrc, pdst, src, dst)

    be2 = b_e2.reshape(1, D)
    e_out, ecol = _edge_call(g_all, edge_feat, W_ef, W_e2, c_e, be2)

    hpart = _scatter_call(e_out, dst, jnp.zeros((N, D), jnp.float32))

    n_out, u_out = _node_call(
        node_feat, hpart, g_repr, ecol, W_nf, W_nh, c_n, W_n2,
        b_n2.reshape(1, D), W_un, W_ue, W_ug, b_u1.reshape(1, D), W_u2,
        b_u2.reshape(1, D))
    return (e_out, n_out, u_out)
